# baseline (speedup 1.0000x reference)
import math

import jax
import jax.numpy as jnp
from jax.experimental import pallas as pl
from jax.experimental.pallas import tpu as pltpu

_PREC = jax.lax.Precision.DEFAULT
_PACK_W = 128


def _dense_taps(w, groups):
    cout, cin_pg = int(w.shape[0]), int(w.shape[1])
    cout_pg = cout // groups
    wt = jnp.transpose(w.reshape(cout, cin_pg, 9), (2, 0, 1))
    grp = jnp.arange(cout) // cout_pg
    onehot = (grp[:, None] == jnp.arange(groups)[None, :]).astype(w.dtype)
    dense = wt[:, :, None, :] * onehot[None, :, :, None]
    return dense.reshape(9, cout, groups * cin_pg)


def kernel(x, init_conv, l0_weight_a, l0_weight_b, l0_bias, l0_c, l0_eta, l0_simc0_conv1, l0_simc0_conv2, l0_simc0_conv3, l0_simc1_conv1, l0_simc1_conv2, l0_simc1_conv3, l0_simc2_conv1, l0_simc2_conv2, l0_simc2_conv3, l0_simc3_conv1, l0_simc3_conv2, l0_simc3_conv3, l0_mimc0_conv1, l0_mimc0_conv2, l0_mimc0_conv3, l0_mimc1_conv1, l0_mimc1_conv2, l0_mimc1_conv3, l0_mimc2_conv1, l0_mimc2_conv2, l0_mimc2_conv3, l1_weight_a, l1_weight_b, l1_bias, l1_c, l1_eta, l1_simc0_conv1, l1_simc0_conv2, l1_simc0_conv3, l1_simc1_conv1, l1_simc1_conv2, l1_simc1_conv3, l1_simc2_conv1, l1_simc2_conv2, l1_simc2_conv3, l1_simc3_conv1, l1_simc3_conv2, l1_simc3_conv3, l1_mimc0_conv1, l1_mimc0_conv2, l1_mimc0_conv3, l1_mimc1_conv1, l1_mimc1_conv2, l1_mimc1_conv3, l1_mimc2_conv1, l1_mimc2_conv2, l1_mimc2_conv3):
    B, Cx, H, W = (int(d) for d in x.shape)
    HW = H * W
    M = int(init_conv.shape[0])
    K = int(l0_weight_a.shape[-1])
    x_flat = x.reshape(B, Cx, HW)

    simc = [
        [{"conv1": l0_simc0_conv1, "conv2": l0_simc0_conv2, "conv3": l0_simc0_conv3},
         {"conv1": l0_simc1_conv1, "conv2": l0_simc1_conv2, "conv3": l0_simc1_conv3},
         {"conv1": l0_simc2_conv1, "conv2": l0_simc2_conv2, "conv3": l0_simc2_conv3},
         {"conv1": l0_simc3_conv1, "conv2": l0_simc3_conv2, "conv3": l0_simc3_conv3}],
        [{"conv1": l1_simc0_conv1, "conv2": l1_simc0_conv2, "conv3": l1_simc0_conv3},
         {"conv1": l1_simc1_conv1, "conv2": l1_simc1_conv2, "conv3": l1_simc1_conv3},
         {"conv1": l1_simc2_conv1, "conv2": l1_simc2_conv2, "conv3": l1_simc2_conv3},
         {"conv1": l1_simc3_conv1, "conv2": l1_simc3_conv2, "conv3": l1_simc3_conv3}],
    ]
    mimc = [
        [{"conv1": l0_mimc0_conv1, "conv2": l0_mimc0_conv2, "conv3": l0_mimc0_conv3},
         {"conv1": l0_mimc1_conv1, "conv2": l0_mimc1_conv2, "conv3": l0_mimc1_conv3},
         {"conv1": l0_mimc2_conv1, "conv2": l0_mimc2_conv2, "conv3": l0_mimc2_conv3}],
        [{"conv1": l1_mimc0_conv1, "conv2": l1_mimc0_conv2, "conv3": l1_mimc0_conv3},
         {"conv1": l1_mimc1_conv1, "conv2": l1_mimc1_conv2, "conv3": l1_mimc1_conv3},
         {"conv1": l1_mimc2_conv1, "conv2": l1_mimc2_conv2, "conv3": l1_mimc2_conv3}],
    ]

    mats = []
    meta_off = [0]

    def _add(m):
        rows, cols = int(m.shape[0]), int(m.shape[1])
        r8 = -(-rows // 8) * 8
        mats.append(jnp.pad(m, ((0, r8 - rows), (0, _PACK_W - cols))))
        off = meta_off[0]
        meta_off[0] += r8
        return (off, rows, cols)

    def _cascade(convs, groups, cin, fold=None):
        piece_ch = [cin]
        stages = []
        for idx, (w, g) in enumerate(zip(convs, groups)):
            ct = sum(piece_ch)
            dense = _dense_taps(w, g)
            assert int(dense.shape[2]) == ct
            dys = []
            for ky in range(3):
                wk = jnp.concatenate(
                    [dense[ky * 3 + 0], dense[ky * 3 + 1], dense[ky * 3 + 2]], axis=1)
                if fold is not None and idx == len(convs) - 1:
                    wk = fold @ wk
                dys.append(_add(wk))
            stages.append(tuple(dys))
            piece_ch.append(int(w.shape[0]))
        return tuple(stages)

    S = (jnp.arange(M)[None, :] // (M // 3) == jnp.arange(3)[:, None]).astype(jnp.float32)
    sig_vecs = [jnp.ones((M,), jnp.float32), l0_c]
    g1 = (1, 3, 6)
    g2 = (4, 8, 12)

    meta_init = _cascade((init_conv,), (3,), Cx)
    meta_A, meta_B, meta_s3, meta_s2, meta_se, meta_me = [], [], [], [], [], []
    for li in range(2):
        sv = sig_vecs[li]
        fold_s = S
        fold_ss = S * sv[None, :]
        mset = lambda j: (mimc[li][j]["conv1"], mimc[li][j]["conv2"], mimc[li][j]["conv3"])
        sset = lambda j: (simc[li][j]["conv1"], simc[li][j]["conv2"], simc[li][j]["conv3"])
        meta_A.append(_cascade(mset(2), g2, M, fold=fold_s))
        meta_B.append(_cascade(mset(2), g2, M, fold=fold_ss))
        meta_s3.append(_cascade(sset(3), g1, Cx))
        meta_s2.append(_cascade(sset(2), g1, Cx))
        meta_se.append([_cascade(sset(e), g1, Cx) for e in range(2)])
        meta_me.append([_cascade(mset(e), g2, M, fold=fold_ss) for e in range(2)])

    packed = jnp.concatenate(mats, axis=0).astype(jnp.bfloat16)
    Rtot = int(packed.shape[0])

    wa_e = [jnp.transpose(l0_weight_a, (0, 2, 1))[..., None],
            jnp.transpose(l1_weight_a, (0, 2, 1))[..., None]]
    wb_e = [jnp.transpose(l0_weight_b, (0, 2, 1))[..., None],
            jnp.transpose(l1_weight_b, (0, 2, 1))[..., None]]
    bias_e = [l0_bias[..., None], l1_bias[..., None]]
    sig1 = l0_c.reshape(M, 1)
    eta_all = jnp.stack([l0_eta, l1_eta])
    n2 = int(l0_eta.shape[0])

    def _body(pk_ref, wa0_ref, wb0_ref, b0_ref, wa1_ref, wb1_ref, b1_ref,
              sig_ref, eta_ref, x_ref, o_ref):
        lane = jax.lax.broadcasted_iota(jnp.int32, (1, HW), 1) % W
        left_edge = lane == 0
        right_edge = lane == W - 1

        def shifted(t):
            r1 = pltpu.roll(t, 1, axis=1)
            rm1 = pltpu.roll(t, HW - 1, axis=1)
            tl = jnp.where(left_edge, pltpu.roll(r1, HW - W, axis=1), r1)
            tr = jnp.where(right_edge, pltpu.roll(rm1, W, axis=1), rm1)
            return tl, tr

        def run_cascade(meta, x0):
            x0 = x0.astype(jnp.bfloat16)
            xl, xr = shifted(x0)
            pl_, pc_, pr_ = [xl], [x0], [xr]
            out = None
            for si, st in enumerate(meta):
                x3 = jnp.concatenate(pl_ + pc_ + pr_, axis=0)
                ys = []
                for dy, (off, rows, cols) in zip((-1, 0, 1), st):
                    wm = pk_ref[off:off + rows, 0:cols]
                    y = jnp.dot(wm, x3, preferred_element_type=jnp.float32,
                                precision=_PREC)
                    shift = (-dy * W) % HW
                    ys.append(y if shift == 0 else pltpu.roll(y, shift, axis=1))
                out = ys[0] + ys[1] + ys[2]
                if si < len(meta) - 1:
                    ob = out.astype(jnp.bfloat16)
                    ol, orr = shifted(ob)
                    pl_.append(ol); pc_.append(ob); pr_.append(orr)
            return out

        def pwl2(wa_ref, wb_ref, bias_ref, xin, need_second):
            s1 = jnp.sin(math.pi * xin)
            c1 = jnp.cos(math.pi * xin)
            s, c = s1, c1
            a0 = wa_ref[0, 0] * s + wb_ref[0, 0] * c
            a1 = wa_ref[1, 0] * s + wb_ref[1, 0] * c if need_second else None
            for k in range(1, K):
                s, c = s * c1 + c * s1, c * c1 - s * s1
                a0 = a0 + wa_ref[0, k] * s + wb_ref[0, k] * c
                if need_second:
                    a1 = a1 + wa_ref[1, k] * s + wb_ref[1, k] * c
            return (a0 + bias_ref[0],
                    a1 + bias_ref[1] if need_second else None)

        xv = x_ref[...]
        z = xv
        u = xv
        lam = run_cascade(meta_init, xv)
        pwl_refs = [(wa0_ref, wb0_ref, b0_ref), (wa1_ref, wb1_ref, b1_ref)]
        for li in range(2):
            wa_r, wb_r, b_r = pwl_refs[li]
            sig = None if li == 0 else sig_ref[...]
            lam_div = lam if li == 0 else lam / sig
            a_term = run_cascade(meta_A[li], lam)
            for e in range(n2):
                f_u = run_cascade(meta_s3[li], u)
                th, dri = pwl2(wa_r, wb_r, b_r, lam_div + f_u, True)
                b_term = run_cascade(meta_B[li], f_u - th)
                out = a_term + b_term + u - z
                f1 = run_cascade(meta_se[li][e], out)
                g2_t = run_cascade(meta_me[li][e], f1 * (1.0 - dri))
                grad = g2_t + out
                u = u - eta_ref[li:li + 1, e:e + 1] * grad
            f_o = run_cascade(meta_s3[li], u)
            p_out, _ = pwl2(wa_r, wb_r, b_r, lam_div + f_o, False)
            m_t = run_cascade(meta_s2[li], u)
            dlam = m_t - p_out
            lam = lam + (dlam if li == 0 else sig * dlam)
        o_ref[...] = jnp.clip(u, 0.0, 1.0)

    full = lambda *dims: pl.BlockSpec(dims, lambda b: (0,) * len(dims))
    out = pl.pallas_call(
        _body,
        out_shape=jax.ShapeDtypeStruct((B, Cx, HW), jnp.float32),
        grid_spec=pltpu.PrefetchScalarGridSpec(
            num_scalar_prefetch=0,
            grid=(B,),
            in_specs=[
                full(Rtot, _PACK_W),
                full(2, K, M, 1), full(2, K, M, 1), full(2, M, 1),
                full(2, K, M, 1), full(2, K, M, 1), full(2, M, 1),
                full(M, 1), full(2, n2),
                pl.BlockSpec((None, Cx, HW), lambda b: (b, 0, 0)),
            ],
            out_specs=pl.BlockSpec((None, Cx, HW), lambda b: (b, 0, 0)),
        ),
        compiler_params=pltpu.CompilerParams(dimension_semantics=("parallel",)),
    )(packed, wa_e[0], wb_e[0], bias_e[0], wa_e[1], wb_e[1], bias_e[1],
      sig1, eta_all, x_flat)
    return out.reshape(B, Cx, H, W)


# speedup vs baseline: 1.0949x; 1.0949x over previous
import math

import jax
import jax.numpy as jnp
from jax.experimental import pallas as pl
from jax.experimental.pallas import tpu as pltpu

_PREC = jax.lax.Precision.DEFAULT
_PACK_W = 128


def _dense_taps(w, groups):
    cout, cin_pg = int(w.shape[0]), int(w.shape[1])
    cout_pg = cout // groups
    wt = jnp.transpose(w.reshape(cout, cin_pg, 9), (2, 0, 1))
    grp = jnp.arange(cout) // cout_pg
    onehot = (grp[:, None] == jnp.arange(groups)[None, :]).astype(w.dtype)
    dense = wt[:, :, None, :] * onehot[None, :, :, None]
    return dense.reshape(9, cout, groups * cin_pg)


def kernel(x, init_conv, l0_weight_a, l0_weight_b, l0_bias, l0_c, l0_eta, l0_simc0_conv1, l0_simc0_conv2, l0_simc0_conv3, l0_simc1_conv1, l0_simc1_conv2, l0_simc1_conv3, l0_simc2_conv1, l0_simc2_conv2, l0_simc2_conv3, l0_simc3_conv1, l0_simc3_conv2, l0_simc3_conv3, l0_mimc0_conv1, l0_mimc0_conv2, l0_mimc0_conv3, l0_mimc1_conv1, l0_mimc1_conv2, l0_mimc1_conv3, l0_mimc2_conv1, l0_mimc2_conv2, l0_mimc2_conv3, l1_weight_a, l1_weight_b, l1_bias, l1_c, l1_eta, l1_simc0_conv1, l1_simc0_conv2, l1_simc0_conv3, l1_simc1_conv1, l1_simc1_conv2, l1_simc1_conv3, l1_simc2_conv1, l1_simc2_conv2, l1_simc2_conv3, l1_simc3_conv1, l1_simc3_conv2, l1_simc3_conv3, l1_mimc0_conv1, l1_mimc0_conv2, l1_mimc0_conv3, l1_mimc1_conv1, l1_mimc1_conv2, l1_mimc1_conv3, l1_mimc2_conv1, l1_mimc2_conv2, l1_mimc2_conv3):
    B, Cx, H, W = (int(d) for d in x.shape)
    HW = H * W
    M = int(init_conv.shape[0])
    K = int(l0_weight_a.shape[-1])
    PAIR = 2 if B % 2 == 0 else 1
    HW2 = PAIR * HW
    W2 = PAIR * W
    x_flat = (x.reshape(B // PAIR, PAIR, Cx, H, W)
              .transpose(0, 2, 3, 1, 4)
              .reshape(B // PAIR, Cx, HW2))

    simc = [
        [{"conv1": l0_simc0_conv1, "conv2": l0_simc0_conv2, "conv3": l0_simc0_conv3},
         {"conv1": l0_simc1_conv1, "conv2": l0_simc1_conv2, "conv3": l0_simc1_conv3},
         {"conv1": l0_simc2_conv1, "conv2": l0_simc2_conv2, "conv3": l0_simc2_conv3},
         {"conv1": l0_simc3_conv1, "conv2": l0_simc3_conv2, "conv3": l0_simc3_conv3}],
        [{"conv1": l1_simc0_conv1, "conv2": l1_simc0_conv2, "conv3": l1_simc0_conv3},
         {"conv1": l1_simc1_conv1, "conv2": l1_simc1_conv2, "conv3": l1_simc1_conv3},
         {"conv1": l1_simc2_conv1, "conv2": l1_simc2_conv2, "conv3": l1_simc2_conv3},
         {"conv1": l1_simc3_conv1, "conv2": l1_simc3_conv2, "conv3": l1_simc3_conv3}],
    ]
    mimc = [
        [{"conv1": l0_mimc0_conv1, "conv2": l0_mimc0_conv2, "conv3": l0_mimc0_conv3},
         {"conv1": l0_mimc1_conv1, "conv2": l0_mimc1_conv2, "conv3": l0_mimc1_conv3},
         {"conv1": l0_mimc2_conv1, "conv2": l0_mimc2_conv2, "conv3": l0_mimc2_conv3}],
        [{"conv1": l1_mimc0_conv1, "conv2": l1_mimc0_conv2, "conv3": l1_mimc0_conv3},
         {"conv1": l1_mimc1_conv1, "conv2": l1_mimc1_conv2, "conv3": l1_mimc1_conv3},
         {"conv1": l1_mimc2_conv1, "conv2": l1_mimc2_conv2, "conv3": l1_mimc2_conv3}],
    ]

    mats = []
    meta_off = [0]

    def _add(m):
        rows, cols = int(m.shape[0]), int(m.shape[1])
        r8 = -(-rows // 8) * 8
        mats.append(jnp.pad(m, ((0, r8 - rows), (0, _PACK_W - cols))))
        off = meta_off[0]
        meta_off[0] += r8
        return (off, rows, cols)

    def _cascade(convs, groups, cin, fold=None):
        piece_ch = [cin]
        stages = []
        for idx, (w, g) in enumerate(zip(convs, groups)):
            ct = sum(piece_ch)
            dense = _dense_taps(w, g)
            assert int(dense.shape[2]) == ct
            dys = []
            for ky in range(3):
                wk = jnp.concatenate(
                    [dense[ky * 3 + 0], dense[ky * 3 + 1], dense[ky * 3 + 2]], axis=1)
                if fold is not None and idx == len(convs) - 1:
                    wk = fold @ wk
                dys.append(_add(wk))
            stages.append(tuple(dys))
            piece_ch.append(int(w.shape[0]))
        return tuple(stages)

    S = (jnp.arange(M)[None, :] // (M // 3) == jnp.arange(3)[:, None]).astype(jnp.float32)
    sig_vecs = [jnp.ones((M,), jnp.float32), l0_c]
    g1 = (1, 3, 6)
    g2 = (4, 8, 12)

    meta_init = _cascade((init_conv,), (3,), Cx)
    meta_A, meta_B, meta_s3, meta_s2, meta_se, meta_me = [], [], [], [], [], []
    for li in range(2):
        sv = sig_vecs[li]
        fold_s = S
        fold_ss = S * sv[None, :]
        mset = lambda j: (mimc[li][j]["conv1"], mimc[li][j]["conv2"], mimc[li][j]["conv3"])
        sset = lambda j: (simc[li][j]["conv1"], simc[li][j]["conv2"], simc[li][j]["conv3"])
        meta_A.append(_cascade(mset(2), g2, M, fold=fold_s))
        meta_B.append(_cascade(mset(2), g2, M, fold=fold_ss))
        meta_s3.append(_cascade(sset(3), g1, Cx))
        meta_s2.append(_cascade(sset(2), g1, Cx))
        meta_se.append([_cascade(sset(e), g1, Cx) for e in range(2)])
        meta_me.append([_cascade(mset(e), g2, M, fold=fold_ss) for e in range(2)])

    packed = jnp.concatenate(mats, axis=0)
    Rtot = int(packed.shape[0])

    wa_e = [jnp.transpose(l0_weight_a, (0, 2, 1))[..., None],
            jnp.transpose(l1_weight_a, (0, 2, 1))[..., None]]
    wb_e = [jnp.transpose(l0_weight_b, (0, 2, 1))[..., None],
            jnp.transpose(l1_weight_b, (0, 2, 1))[..., None]]
    bias_e = [l0_bias[..., None], l1_bias[..., None]]
    sig1 = l0_c.reshape(M, 1)
    eta_all = jnp.stack([l0_eta, l1_eta])
    n2 = int(l0_eta.shape[0])

    def _body(pk_ref, wa0_ref, wb0_ref, b0_ref, wa1_ref, wb1_ref, b1_ref,
              sig_ref, eta_ref, x_ref, o_ref):
        lane = jax.lax.broadcasted_iota(jnp.int32, (1, HW2), 1) % W
        left_edge = lane == 0
        right_edge = lane == W - 1

        def shifted(t):
            r1 = pltpu.roll(t, 1, axis=1)
            rm1 = pltpu.roll(t, HW2 - 1, axis=1)
            tl = jnp.where(left_edge, pltpu.roll(r1, HW2 - W, axis=1), r1)
            tr = jnp.where(right_edge, pltpu.roll(rm1, W, axis=1), rm1)
            return tl, tr

        def run_cascade(meta, x0):
            xl, xr = shifted(x0)
            pl_, pc_, pr_ = [xl], [x0], [xr]
            out = None
            for si, st in enumerate(meta):
                x3 = jnp.concatenate(pl_ + pc_ + pr_, axis=0)
                ys = []
                for dy, (off, rows, cols) in zip((-1, 0, 1), st):
                    wm = pk_ref[off:off + rows, 0:cols]
                    y = jnp.dot(wm, x3, preferred_element_type=jnp.float32,
                                precision=_PREC)
                    shift = (-dy * W2) % HW2
                    ys.append(y if shift == 0 else pltpu.roll(y, shift, axis=1))
                out = ys[0] + ys[1] + ys[2]
                if si < len(meta) - 1:
                    ol, orr = shifted(out)
                    pl_.append(ol); pc_.append(out); pr_.append(orr)
            return out

        def pwl2(wa_ref, wb_ref, bias_ref, xin, need_second):
            s1 = jnp.sin(math.pi * xin)
            c1 = jnp.cos(math.pi * xin)
            s, c = s1, c1
            a0 = wa_ref[0, 0] * s + wb_ref[0, 0] * c
            a1 = wa_ref[1, 0] * s + wb_ref[1, 0] * c if need_second else None
            for k in range(1, K):
                s, c = s * c1 + c * s1, c * c1 - s * s1
                a0 = a0 + wa_ref[0, k] * s + wb_ref[0, k] * c
                if need_second:
                    a1 = a1 + wa_ref[1, k] * s + wb_ref[1, k] * c
            return (a0 + bias_ref[0],
                    a1 + bias_ref[1] if need_second else None)

        xv = x_ref[...]
        z = xv
        u = xv
        lam = run_cascade(meta_init, xv)
        pwl_refs = [(wa0_ref, wb0_ref, b0_ref), (wa1_ref, wb1_ref, b1_ref)]
        for li in range(2):
            wa_r, wb_r, b_r = pwl_refs[li]
            sig = None if li == 0 else sig_ref[...]
            lam_div = lam if li == 0 else lam / sig
            a_term = run_cascade(meta_A[li], lam)
            for e in range(n2):
                f_u = run_cascade(meta_s3[li], u)
                th, dri = pwl2(wa_r, wb_r, b_r, lam_div + f_u, True)
                b_term = run_cascade(meta_B[li], f_u - th)
                out = a_term + b_term + u - z
                f1 = run_cascade(meta_se[li][e], out)
                g2_t = run_cascade(meta_me[li][e], f1 * (1.0 - dri))
                grad = g2_t + out
                u = u - eta_ref[li:li + 1, e:e + 1] * grad
            f_o = run_cascade(meta_s3[li], u)
            p_out, _ = pwl2(wa_r, wb_r, b_r, lam_div + f_o, False)
            m_t = run_cascade(meta_s2[li], u)
            dlam = m_t - p_out
            lam = lam + (dlam if li == 0 else sig * dlam)
        o_ref[...] = jnp.clip(u, 0.0, 1.0)

    full = lambda *dims: pl.BlockSpec(dims, lambda b: (0,) * len(dims))
    out = pl.pallas_call(
        _body,
        out_shape=jax.ShapeDtypeStruct((B // PAIR, Cx, HW2), jnp.float32),
        grid_spec=pltpu.PrefetchScalarGridSpec(
            num_scalar_prefetch=0,
            grid=(B // PAIR,),
            in_specs=[
                full(Rtot, _PACK_W),
                full(2, K, M, 1), full(2, K, M, 1), full(2, M, 1),
                full(2, K, M, 1), full(2, K, M, 1), full(2, M, 1),
                full(M, 1), full(2, n2),
                pl.BlockSpec((None, Cx, HW2), lambda b: (b, 0, 0)),
            ],
            out_specs=pl.BlockSpec((None, Cx, HW2), lambda b: (b, 0, 0)),
        ),
        compiler_params=pltpu.CompilerParams(dimension_semantics=("parallel",)),
    )(packed, wa_e[0], wb_e[0], bias_e[0], wa_e[1], wb_e[1], bias_e[1],
      sig1, eta_all, x_flat)
    return (out.reshape(B // PAIR, Cx, H, PAIR, W)
            .transpose(0, 3, 1, 2, 4)
            .reshape(B, Cx, H, W))


# speedup vs baseline: 1.1783x; 1.0762x over previous
import math

import jax
import jax.numpy as jnp
from jax.experimental import pallas as pl
from jax.experimental.pallas import tpu as pltpu

_PREC = jax.lax.Precision.DEFAULT
_PACK_W = 128


def _dense_taps(w, groups):
    cout, cin_pg = int(w.shape[0]), int(w.shape[1])
    cout_pg = cout // groups
    wt = jnp.transpose(w.reshape(cout, cin_pg, 9), (2, 0, 1))
    grp = jnp.arange(cout) // cout_pg
    onehot = (grp[:, None] == jnp.arange(groups)[None, :]).astype(w.dtype)
    dense = wt[:, :, None, :] * onehot[None, :, :, None]
    return dense.reshape(9, cout, groups * cin_pg)


def kernel(x, init_conv, l0_weight_a, l0_weight_b, l0_bias, l0_c, l0_eta, l0_simc0_conv1, l0_simc0_conv2, l0_simc0_conv3, l0_simc1_conv1, l0_simc1_conv2, l0_simc1_conv3, l0_simc2_conv1, l0_simc2_conv2, l0_simc2_conv3, l0_simc3_conv1, l0_simc3_conv2, l0_simc3_conv3, l0_mimc0_conv1, l0_mimc0_conv2, l0_mimc0_conv3, l0_mimc1_conv1, l0_mimc1_conv2, l0_mimc1_conv3, l0_mimc2_conv1, l0_mimc2_conv2, l0_mimc2_conv3, l1_weight_a, l1_weight_b, l1_bias, l1_c, l1_eta, l1_simc0_conv1, l1_simc0_conv2, l1_simc0_conv3, l1_simc1_conv1, l1_simc1_conv2, l1_simc1_conv3, l1_simc2_conv1, l1_simc2_conv2, l1_simc2_conv3, l1_simc3_conv1, l1_simc3_conv2, l1_simc3_conv3, l1_mimc0_conv1, l1_mimc0_conv2, l1_mimc0_conv3, l1_mimc1_conv1, l1_mimc1_conv2, l1_mimc1_conv3, l1_mimc2_conv1, l1_mimc2_conv2, l1_mimc2_conv3):
    B, Cx, H, W = (int(d) for d in x.shape)
    HW = H * W
    M = int(init_conv.shape[0])
    K = int(l0_weight_a.shape[-1])
    PAIR = 2 if B % 2 == 0 else 1
    HW2 = PAIR * HW
    W2 = PAIR * W
    x_flat = (x.reshape(B // PAIR, PAIR, Cx, H, W)
              .transpose(0, 2, 3, 1, 4)
              .reshape(B // PAIR, Cx, HW2))

    simc = [
        [{"conv1": l0_simc0_conv1, "conv2": l0_simc0_conv2, "conv3": l0_simc0_conv3},
         {"conv1": l0_simc1_conv1, "conv2": l0_simc1_conv2, "conv3": l0_simc1_conv3},
         {"conv1": l0_simc2_conv1, "conv2": l0_simc2_conv2, "conv3": l0_simc2_conv3},
         {"conv1": l0_simc3_conv1, "conv2": l0_simc3_conv2, "conv3": l0_simc3_conv3}],
        [{"conv1": l1_simc0_conv1, "conv2": l1_simc0_conv2, "conv3": l1_simc0_conv3},
         {"conv1": l1_simc1_conv1, "conv2": l1_simc1_conv2, "conv3": l1_simc1_conv3},
         {"conv1": l1_simc2_conv1, "conv2": l1_simc2_conv2, "conv3": l1_simc2_conv3},
         {"conv1": l1_simc3_conv1, "conv2": l1_simc3_conv2, "conv3": l1_simc3_conv3}],
    ]
    mimc = [
        [{"conv1": l0_mimc0_conv1, "conv2": l0_mimc0_conv2, "conv3": l0_mimc0_conv3},
         {"conv1": l0_mimc1_conv1, "conv2": l0_mimc1_conv2, "conv3": l0_mimc1_conv3},
         {"conv1": l0_mimc2_conv1, "conv2": l0_mimc2_conv2, "conv3": l0_mimc2_conv3}],
        [{"conv1": l1_mimc0_conv1, "conv2": l1_mimc0_conv2, "conv3": l1_mimc0_conv3},
         {"conv1": l1_mimc1_conv1, "conv2": l1_mimc1_conv2, "conv3": l1_mimc1_conv3},
         {"conv1": l1_mimc2_conv1, "conv2": l1_mimc2_conv2, "conv3": l1_mimc2_conv3}],
    ]

    mats = []
    meta_off = [0]

    def _add(blocks):
        rows, cols = int(blocks[0].shape[0]), int(blocks[0].shape[1])
        r8 = -(-rows // 8) * 8
        for m in blocks:
            mats.append(jnp.pad(m, ((0, r8 - rows), (0, _PACK_W - cols))))
        off = meta_off[0]
        meta_off[0] += 3 * r8
        return (off, r8, rows, cols)

    def _cascade(convs, groups, cin, fold=None):
        piece_ch = [cin]
        stages = []
        for idx, (w, g) in enumerate(zip(convs, groups)):
            ct = sum(piece_ch)
            dense = _dense_taps(w, g)
            assert int(dense.shape[2]) == ct
            blocks = []
            for ky in range(3):
                wk = jnp.concatenate(
                    [dense[ky * 3 + 0], dense[ky * 3 + 1], dense[ky * 3 + 2]], axis=1)
                if fold is not None and idx == len(convs) - 1:
                    wk = fold @ wk
                blocks.append(wk)
            stages.append(_add(blocks))
            piece_ch.append(int(w.shape[0]))
        return tuple(stages)

    S = (jnp.arange(M)[None, :] // (M // 3) == jnp.arange(3)[:, None]).astype(jnp.float32)
    sig_vecs = [jnp.ones((M,), jnp.float32), l0_c]
    g1 = (1, 3, 6)
    g2 = (4, 8, 12)

    meta_init = _cascade((init_conv,), (3,), Cx)
    meta_A, meta_B, meta_s3, meta_s2, meta_se, meta_me = [], [], [], [], [], []
    for li in range(2):
        sv = sig_vecs[li]
        fold_s = S
        fold_ss = S * sv[None, :]
        mset = lambda j: (mimc[li][j]["conv1"], mimc[li][j]["conv2"], mimc[li][j]["conv3"])
        sset = lambda j: (simc[li][j]["conv1"], simc[li][j]["conv2"], simc[li][j]["conv3"])
        meta_A.append(_cascade(mset(2), g2, M, fold=fold_s))
        meta_B.append(_cascade(mset(2), g2, M, fold=fold_ss))
        meta_s3.append(_cascade(sset(3), g1, Cx))
        meta_s2.append(_cascade(sset(2), g1, Cx))
        meta_se.append([_cascade(sset(e), g1, Cx) for e in range(2)])
        meta_me.append([_cascade(mset(e), g2, M, fold=fold_ss) for e in range(2)])

    packed = jnp.concatenate(mats, axis=0)
    Rtot = int(packed.shape[0])

    wa_e = [jnp.transpose(l0_weight_a, (0, 2, 1))[..., None],
            jnp.transpose(l1_weight_a, (0, 2, 1))[..., None]]
    wb_e = [jnp.transpose(l0_weight_b, (0, 2, 1))[..., None],
            jnp.transpose(l1_weight_b, (0, 2, 1))[..., None]]
    bias_e = [l0_bias[..., None], l1_bias[..., None]]
    sig1 = l0_c.reshape(M, 1)
    eta_all = jnp.stack([l0_eta, l1_eta])
    n2 = int(l0_eta.shape[0])

    def _body(pk_ref, wa0_ref, wb0_ref, b0_ref, wa1_ref, wb1_ref, b1_ref,
              sig_ref, eta_ref, x_ref, o_ref):
        lane = jax.lax.broadcasted_iota(jnp.int32, (1, HW2), 1) % W
        left_edge = lane == 0
        right_edge = lane == W - 1

        def shifted(t):
            r1 = pltpu.roll(t, 1, axis=1)
            rm1 = pltpu.roll(t, HW2 - 1, axis=1)
            tl = jnp.where(left_edge, pltpu.roll(r1, HW2 - W, axis=1), r1)
            tr = jnp.where(right_edge, pltpu.roll(rm1, W, axis=1), rm1)
            return tl, tr

        def run_cascade(meta, x0):
            xl, xr = shifted(x0)
            pl_, pc_, pr_ = [xl], [x0], [xr]
            out = None
            for si, (off, r8, rows, cols) in enumerate(meta):
                x3 = jnp.concatenate(pl_ + pc_ + pr_, axis=0)
                wm = pk_ref[off:off + 3 * r8, 0:cols]
                y = jnp.dot(wm, x3, preferred_element_type=jnp.float32,
                            precision=_PREC)
                out = (pltpu.roll(y[0:rows], W2, axis=1)
                       + y[r8:r8 + rows]
                       + pltpu.roll(y[2 * r8:2 * r8 + rows], HW2 - W2, axis=1))
                if si < len(meta) - 1:
                    ol, orr = shifted(out)
                    pl_.append(ol); pc_.append(out); pr_.append(orr)
            return out

        def pwl2(wa_ref, wb_ref, bias_ref, xin, need_second):
            s1 = jnp.sin(math.pi * xin)
            c1 = jnp.cos(math.pi * xin)
            s, c = s1, c1
            a0 = wa_ref[0, 0] * s + wb_ref[0, 0] * c
            a1 = wa_ref[1, 0] * s + wb_ref[1, 0] * c if need_second else None
            for k in range(1, K):
                s, c = s * c1 + c * s1, c * c1 - s * s1
                a0 = a0 + wa_ref[0, k] * s + wb_ref[0, k] * c
                if need_second:
                    a1 = a1 + wa_ref[1, k] * s + wb_ref[1, k] * c
            return (a0 + bias_ref[0],
                    a1 + bias_ref[1] if need_second else None)

        xv = x_ref[...]
        z = xv
        u = xv
        lam = run_cascade(meta_init, xv)
        pwl_refs = [(wa0_ref, wb0_ref, b0_ref), (wa1_ref, wb1_ref, b1_ref)]
        for li in range(2):
            wa_r, wb_r, b_r = pwl_refs[li]
            sig = None if li == 0 else sig_ref[...]
            lam_div = lam if li == 0 else lam / sig
            a_term = run_cascade(meta_A[li], lam)
            for e in range(n2):
                f_u = run_cascade(meta_s3[li], u)
                th, dri = pwl2(wa_r, wb_r, b_r, lam_div + f_u, True)
                b_term = run_cascade(meta_B[li], f_u - th)
                out = a_term + b_term + u - z
                f1 = run_cascade(meta_se[li][e], out)
                g2_t = run_cascade(meta_me[li][e], f1 * (1.0 - dri))
                grad = g2_t + out
                u = u - eta_ref[li:li + 1, e:e + 1] * grad
            f_o = run_cascade(meta_s3[li], u)
            p_out, _ = pwl2(wa_r, wb_r, b_r, lam_div + f_o, False)
            m_t = run_cascade(meta_s2[li], u)
            dlam = m_t - p_out
            lam = lam + (dlam if li == 0 else sig * dlam)
        o_ref[...] = jnp.clip(u, 0.0, 1.0)

    full = lambda *dims: pl.BlockSpec(dims, lambda b: (0,) * len(dims))
    out = pl.pallas_call(
        _body,
        out_shape=jax.ShapeDtypeStruct((B // PAIR, Cx, HW2), jnp.float32),
        grid_spec=pltpu.PrefetchScalarGridSpec(
            num_scalar_prefetch=0,
            grid=(B // PAIR,),
            in_specs=[
                full(Rtot, _PACK_W),
                full(2, K, M, 1), full(2, K, M, 1), full(2, M, 1),
                full(2, K, M, 1), full(2, K, M, 1), full(2, M, 1),
                full(M, 1), full(2, n2),
                pl.BlockSpec((None, Cx, HW2), lambda b: (b, 0, 0)),
            ],
            out_specs=pl.BlockSpec((None, Cx, HW2), lambda b: (b, 0, 0)),
        ),
        compiler_params=pltpu.CompilerParams(dimension_semantics=("parallel",)),
    )(packed, wa_e[0], wb_e[0], bias_e[0], wa_e[1], wb_e[1], bias_e[1],
      sig1, eta_all, x_flat)
    return (out.reshape(B // PAIR, Cx, H, PAIR, W)
            .transpose(0, 3, 1, 2, 4)
            .reshape(B, Cx, H, W))
